# baseline (speedup 1.0000x reference)
import functools

import jax
import jax.numpy as jnp
from jax import lax
from jax.experimental import pallas as pl
from jax.experimental.pallas import tpu as pltpu
from jax.experimental.pallas import tpu_sc as plsc

N = 10000
E = 320000
IN = 128
H = 8
F1 = 8
OUT = 40
D1 = H * F1
D2 = H * OUT

KS = 128
K = 256
NSUB = 16
CHUNKS = 80
ESUB = CHUNKS * K
EP = ESUB * NSUB
ACC_ROWS = 10112
R_INIT = ACC_ROWS // NSUB
R_OUT = ACC_ROWS // NSUB


def _sc_edge_body(f_half, fd, sect, kk, src_r, dst_r, h_t, ad_t, numi_r,
                  num_o, acc, *bufs):
    n_chunks = ESUB // kk
    c = lax.axis_index("c")
    s = lax.axis_index("s")
    nh_half = f_half // fd
    r0 = s * R_INIT
    pltpu.sync_copy(numi_r.at[pl.ds(c * ACC_ROWS + r0, R_INIT)],
                    acc.at[pl.ds(r0, R_INIT)])
    plsc.subcore_barrier()

    lanes = lax.iota(jnp.int32, 16)
    zero16 = jnp.full((16,), 0, jnp.int32)
    hsplats = [zero16 + (f_half + (2 * sect + c) * nh_half + t)
               for t in range(nh_half)]
    e0 = s * ESUB
    B = [bufs[5 * b:5 * b + 5] for b in range(4)]
    isems = bufs[20:24]
    gsems = bufs[24:28]
    ssems = bufs[28:32]

    def issue_idx(b, ch):
        idx_s, _, idx_d = B[b][0], B[b][1], B[b][2]
        base = e0 // KS + ch * (kk // KS)
        pltpu.async_copy(src_r.at[pl.ds(base, kk // KS)], idx_s, isems[b])
        pltpu.async_copy(dst_r.at[pl.ds(base, kk // KS)], idx_d, isems[b])

    def wait_idx(b):
        idx_s, _, idx_d = B[b][0], B[b][1], B[b][2]
        pltpu.make_async_copy(src_r.at[pl.ds(0, kk // KS)], idx_s,
                              isems[b]).wait()
        pltpu.make_async_copy(dst_r.at[pl.ds(0, kk // KS)], idx_d,
                              isems[b]).wait()

    def issue_gath(b):
        idx_s, idx_hs, idx_d, ad_rows, hs_rows = B[b]
        for j in range(kk // KS):
            for q in range(KS // 16):
                idx_hs[j, pl.ds(16 * q, 16)] = (
                    idx_s[j, pl.ds(16 * q, 16)] + c * N)
        for j in range(kk // KS):
            pltpu.async_copy(ad_t.at[idx_d.at[j]],
                             ad_rows.at[pl.ds(KS * j, KS)], gsems[b])
            pltpu.async_copy(h_t.at[idx_hs.at[j]],
                             hs_rows.at[pl.ds(KS * j, KS)], gsems[b])

    def wait_scat(b):
        _, _, idx_d, _, hs_rows = B[b]
        for j in range(kk // KS):
            pltpu.make_async_copy(hs_rows.at[pl.ds(KS * j, KS)],
                                  acc.at[idx_d.at[j]], ssems[b]).wait()

    def comp(b):
        idx_s, idx_hs, idx_d, ad_rows, hs_rows = B[b]
        for j in range(kk // KS):
            pltpu.make_async_copy(ad_t.at[idx_d.at[j]],
                                  ad_rows.at[pl.ds(KS * j, KS)],
                                  gsems[b]).wait()
            pltpu.make_async_copy(h_t.at[idx_hs.at[j]],
                                  hs_rows.at[pl.ds(KS * j, KS)],
                                  gsems[b]).wait()

        def exq(qq, carry):
            for u in range(8):
                p = 128 * qq + 16 * u + lanes
                row = p // H
                col = p % H
                a = (plsc.load_gather(hs_rows, [row, f_half + col]) +
                     plsc.load_gather(ad_rows, [row, col]))
                a = jnp.where(a > 0, a, 0.2 * a)
                plsc.store_scatter(hs_rows, [row, f_half + col], jnp.exp(a))
            return carry

        lax.fori_loop(0, kk * H // 128, exq, 0)

        def group(g, carry):
            kv = g * 16 + lanes
            ws = [plsc.load_gather(hs_rows, [kv, hs]) for hs in hsplats]
            for f in range(f_half):
                fv = zero16 + f
                hv = plsc.load_gather(hs_rows, [kv, fv])
                plsc.store_scatter(hs_rows, [kv, fv], hv * ws[f // fd])
            return carry

        lax.fori_loop(0, kk // 16, group, 0)
        for j in range(kk // KS):
            pltpu.async_copy(hs_rows.at[pl.ds(KS * j, KS)],
                             acc.at[idx_d.at[j]], ssems[b], add=True)

    issue_idx(0, 0)
    issue_idx(1, 1)
    wait_idx(0)
    issue_gath(0)

    def blk(ib, carry):
        for b in range(4):
            i = 4 * ib + b

            @pl.when(i >= 2)
            def _():
                wait_scat((b + 2) % 4)

            @pl.when(i + 2 < n_chunks)
            def _():
                issue_idx((b + 2) % 4, i + 2)

            @pl.when(i + 1 < n_chunks)
            def _():
                wait_idx((b + 1) % 4)
                issue_gath((b + 1) % 4)

            comp(b)
        return carry

    lax.fori_loop(0, n_chunks // 4, blk, 0)
    wait_scat((n_chunks - 2) % 4)
    wait_scat((n_chunks - 1) % 4)

    plsc.subcore_barrier()
    w0 = s * R_OUT
    pltpu.sync_copy(acc.at[pl.ds(w0, R_OUT)],
                    num_o.at[pl.ds(c * ACC_ROWS + w0, R_OUT)])


@functools.lru_cache(maxsize=None)
def _make_sc_edge(f_half, fd, sect=0, kk=K):
    mesh = plsc.VectorSubcoreMesh(core_axis_name="c", subcore_axis_name="s",
                                  num_cores=2, num_subcores=NSUB)
    fw = f_half + H
    buf_set = [
        pltpu.VMEM((kk // KS, KS), jnp.int32),
        pltpu.VMEM((kk // KS, KS), jnp.int32),
        pltpu.VMEM((kk // KS, KS), jnp.int32),
        pltpu.VMEM((kk, H), jnp.float32),
        pltpu.VMEM((kk, fw), jnp.float32),
    ]
    return pl.kernel(
        functools.partial(_sc_edge_body, f_half, fd, sect, kk),
        compiler_params=pltpu.CompilerParams(use_tc_tiling_on_sc=False,
                                             needs_layout_passes=False),
        out_type=[jax.ShapeDtypeStruct((2 * ACC_ROWS, fw), jnp.float32)],
        mesh=mesh,
        scratch_types=(
            [pltpu.VMEM_SHARED((ACC_ROWS, fw), jnp.float32)]
            + buf_set * 4
            + [pltpu.SemaphoreType.DMA] * 12
        ),
    )


def _tc1_body(x_r, w1_r, asm_r, adm_r, h_o, as_o, ad_o, exs_o, numi_o):
    x = x_r[...]
    h = lax.dot_general(x, w1_r[...], (((1,), (1,)), ((), ())),
                        preferred_element_type=jnp.float32)
    a_s = jnp.dot(h, asm_r[...], preferred_element_type=jnp.float32)
    a_d = jnp.dot(h, adm_r[...], preferred_element_type=jnp.float32)
    al = a_s + a_d
    al = jnp.where(al > 0, al, 0.2 * al)
    exs = jnp.exp(al)
    h_o[...] = h
    as_o[...] = a_s
    ad_o[...] = a_d
    exs_o[...] = exs
    numi_o[...] = jnp.concatenate(
        [h[:, F1 * t:F1 * (t + 1)] * exs[:, t:t + 1] for t in range(H)],
        axis=1)


def _tc2_body(num_r, den_r, b1_r, w2_r, asm_r, adm_r,
              h_o, as_o, ad_o, exs_o, numi_o):
    num = num_r[...]
    r = 1.0 / (den_r[...] + 1e-16)
    h1 = jnp.concatenate(
        [num[:, F1 * t:F1 * (t + 1)] * r[:, t:t + 1] for t in range(H)],
        axis=1) + b1_r[...]
    h1 = jnp.where(h1 > 0, h1, jnp.exp(jnp.minimum(h1, 0.0)) - 1.0)
    h = lax.dot_general(h1, w2_r[...], (((1,), (1,)), ((), ())),
                        preferred_element_type=jnp.float32)
    a_s = jnp.dot(h, asm_r[...], preferred_element_type=jnp.float32)
    a_d = jnp.dot(h, adm_r[...], preferred_element_type=jnp.float32)
    al = a_s + a_d
    al = jnp.where(al > 0, al, 0.2 * al)
    exs = jnp.exp(al)
    h_o[...] = h
    as_o[...] = a_s
    ad_o[...] = a_d
    exs_o[...] = exs
    numi_o[...] = jnp.concatenate(
        [h[:, OUT * t:OUT * (t + 1)] * exs[:, t:t + 1] for t in range(H)],
        axis=1)


def _tc3_body(num_r, den_r, b2_r, out_o):
    num = num_r[...]
    r = 0.125 / (den_r[...] + 1e-16)
    acc = num[:, 0:OUT] * r[:, 0:1]
    for t in range(1, H):
        acc = acc + num[:, OUT * t:OUT * (t + 1)] * r[:, t:t + 1]
    out_o[...] = acc + b2_r[...]


_BN = 2000
_GRID = N // _BN


def _row_spec(width):
    return pl.BlockSpec((_BN, width), lambda i: (i, 0))


def _full_spec(shape):
    return pl.BlockSpec(shape, lambda i: tuple(0 for _ in shape))


_tc1 = pl.pallas_call(
    _tc1_body,
    grid=(_GRID,),
    in_specs=[_row_spec(IN), _full_spec((D1, IN)), _full_spec((D1, H)),
              _full_spec((D1, H))],
    out_specs=[_row_spec(D1), _row_spec(H), _row_spec(H), _row_spec(H),
               _row_spec(D1)],
    out_shape=[jax.ShapeDtypeStruct((N, D1), jnp.float32),
               jax.ShapeDtypeStruct((N, H), jnp.float32),
               jax.ShapeDtypeStruct((N, H), jnp.float32),
               jax.ShapeDtypeStruct((N, H), jnp.float32),
               jax.ShapeDtypeStruct((N, D1), jnp.float32)],
)

_tc2 = pl.pallas_call(
    _tc2_body,
    grid=(_GRID,),
    in_specs=[_row_spec(D1), _row_spec(H), _full_spec((1, D1)),
              _full_spec((D2, D1)), _full_spec((D2, H)), _full_spec((D2, H))],
    out_specs=[_row_spec(D2), _row_spec(H), _row_spec(H), _row_spec(H),
               _row_spec(D2)],
    out_shape=[jax.ShapeDtypeStruct((N, D2), jnp.float32),
               jax.ShapeDtypeStruct((N, H), jnp.float32),
               jax.ShapeDtypeStruct((N, H), jnp.float32),
               jax.ShapeDtypeStruct((N, H), jnp.float32),
               jax.ShapeDtypeStruct((N, D2), jnp.float32)],
)

_tc3 = pl.pallas_call(
    _tc3_body,
    grid=(_GRID,),
    in_specs=[_row_spec(D2), _row_spec(H), _full_spec((1, OUT))],
    out_specs=_row_spec(OUT),
    out_shape=jax.ShapeDtypeStruct((N, OUT), jnp.float32),
)


def _expand_att(a):
    hh, ff = a.shape
    eye = jnp.eye(hh, dtype=jnp.float32)
    return (a[:, :, None] * eye[:, None, :]).reshape(hh * ff, hh)


def kernel(x, edge, W1, a_src1, a_dst1, b1, W2, a_src2, a_dst2, b2):
    As1 = _expand_att(a_src1)
    Ad1 = _expand_att(a_dst1)
    As2 = _expand_att(a_src2)
    Ad2 = _expand_att(a_dst2)
    src = jnp.concatenate(
        [edge[0].astype(jnp.int32),
         jnp.zeros((EP - E,), jnp.int32)]).reshape(EP // KS, KS)
    dst = jnp.concatenate(
        [edge[1].astype(jnp.int32),
         jnp.full((EP - E,), N, jnp.int32)]).reshape(EP // KS, KS)

    h1, as1, ad1, exs1, numi1 = _tc1(x, W1, As1, Ad1)
    fh1 = D1 // 2
    ht1 = jnp.concatenate([
        jnp.concatenate([h1[:, :fh1], as1], axis=1),
        jnp.concatenate([h1[:, fh1:], as1], axis=1)], axis=0)
    ad1p = jnp.concatenate([ad1, jnp.zeros((1, H), jnp.float32)], axis=0)
    pad1 = jnp.zeros((ACC_ROWS - N, fh1 + H), jnp.float32)
    numi1f = jnp.concatenate([
        jnp.concatenate([numi1[:, :fh1], exs1], axis=1), pad1,
        jnp.concatenate([numi1[:, fh1:], exs1], axis=1), pad1], axis=0)
    out1 = _make_sc_edge(fh1, F1, 0)(src, dst, ht1, ad1p, numi1f)
    num1 = out1[0] if isinstance(out1, (tuple, list)) else out1
    num1c = jnp.concatenate(
        [num1[:N, :fh1], num1[ACC_ROWS:ACC_ROWS + N, :fh1]], axis=1)
    den1 = num1[:N, fh1:]

    h2, as2, ad2, exs2, numi2 = _tc2(num1c, den1, b1.reshape(1, D1),
                                     W2, As2, Ad2)
    ad2p = jnp.concatenate([ad2, jnp.zeros((1, H), jnp.float32)], axis=0)
    fq = D2 // 4
    padq = jnp.zeros((ACC_ROWS - N, fq + H), jnp.float32)
    quarters = []
    den2 = None
    for p in range(2):
        ht = jnp.concatenate([
            jnp.concatenate([h2[:, 2 * p * fq:(2 * p + 1) * fq], as2],
                            axis=1),
            jnp.concatenate([h2[:, (2 * p + 1) * fq:(2 * p + 2) * fq], as2],
                            axis=1)], axis=0)
        numib = jnp.concatenate([
            jnp.concatenate([numi2[:, 2 * p * fq:(2 * p + 1) * fq], exs2],
                            axis=1), padq,
            jnp.concatenate([numi2[:, (2 * p + 1) * fq:(2 * p + 2) * fq],
                             exs2], axis=1), padq], axis=0)
        outq = _make_sc_edge(fq, OUT, p, KS)(src, dst, ht, ad2p, numib)
        numq = outq[0] if isinstance(outq, (tuple, list)) else outq
        quarters += [numq[:N, :fq], numq[ACC_ROWS:ACC_ROWS + N, :fq]]
        if p == 0:
            den2 = numq[:N, fq:]
    num2c = jnp.concatenate(quarters, axis=1)

    return _tc3(num2c, den2, b2.reshape(1, OUT))


# speedup vs baseline: 1.0008x; 1.0008x over previous
import functools

import jax
import jax.numpy as jnp
from jax import lax
from jax.experimental import pallas as pl
from jax.experimental.pallas import tpu as pltpu
from jax.experimental.pallas import tpu_sc as plsc

N = 10000
E = 320000
IN = 128
H = 8
F1 = 8
OUT = 40
D1 = H * F1
D2 = H * OUT

KS = 128
K = 256
NSUB = 16
CHUNKS = 80
ESUB = CHUNKS * K
EP = ESUB * NSUB
ACC_ROWS = 10112
R_INIT = ACC_ROWS // NSUB
R_OUT = ACC_ROWS // NSUB


def _sc_edge_body(f_half, fd, sect, kk, src_r, dst_r, h_t, ad_t, numi_r,
                  num_o, acc, *bufs):
    n_chunks = ESUB // kk
    c = lax.axis_index("c")
    s = lax.axis_index("s")
    nh_half = f_half // fd
    r0 = s * R_INIT
    pltpu.sync_copy(numi_r.at[pl.ds(c * ACC_ROWS + r0, R_INIT)],
                    acc.at[pl.ds(r0, R_INIT)])
    plsc.subcore_barrier()

    lanes = lax.iota(jnp.int32, 16)
    zero16 = jnp.full((16,), 0, jnp.int32)
    hsplats = [zero16 + (f_half + (2 * sect + c) * nh_half + t)
               for t in range(nh_half)]
    e0 = s * ESUB
    B = [bufs[5 * b:5 * b + 5] for b in range(4)]
    isems = bufs[20:24]
    gsems = bufs[24:28]
    ssems = bufs[28:32]

    def issue_idx(b, ch):
        idx_s, _, idx_d = B[b][0], B[b][1], B[b][2]
        base = e0 // KS + ch * (kk // KS)
        pltpu.async_copy(src_r.at[pl.ds(base, kk // KS)], idx_s, isems[b])
        pltpu.async_copy(dst_r.at[pl.ds(base, kk // KS)], idx_d, isems[b])

    def wait_idx(b):
        idx_s, _, idx_d = B[b][0], B[b][1], B[b][2]
        pltpu.make_async_copy(src_r.at[pl.ds(0, kk // KS)], idx_s,
                              isems[b]).wait()
        pltpu.make_async_copy(dst_r.at[pl.ds(0, kk // KS)], idx_d,
                              isems[b]).wait()

    def issue_gath(b):
        idx_s, idx_hs, idx_d, ad_rows, hs_rows = B[b]
        for j in range(kk // KS):
            for q in range(KS // 16):
                idx_hs[j, pl.ds(16 * q, 16)] = (
                    idx_s[j, pl.ds(16 * q, 16)] + c * N)
        for j in range(kk // KS):
            pltpu.async_copy(ad_t.at[idx_d.at[j]],
                             ad_rows.at[pl.ds(KS * j, KS)], gsems[b])
            pltpu.async_copy(h_t.at[idx_hs.at[j]],
                             hs_rows.at[pl.ds(KS * j, KS)], gsems[b])

    def wait_scat(b):
        _, _, idx_d, _, hs_rows = B[b]
        for j in range(kk // KS):
            pltpu.make_async_copy(hs_rows.at[pl.ds(KS * j, KS)],
                                  acc.at[idx_d.at[j]], ssems[b]).wait()

    def comp(b):
        idx_s, idx_hs, idx_d, ad_rows, hs_rows = B[b]
        for j in range(kk // KS):
            pltpu.make_async_copy(ad_t.at[idx_d.at[j]],
                                  ad_rows.at[pl.ds(KS * j, KS)],
                                  gsems[b]).wait()
            pltpu.make_async_copy(h_t.at[idx_hs.at[j]],
                                  hs_rows.at[pl.ds(KS * j, KS)],
                                  gsems[b]).wait()

        def exq(qq, carry):
            for u in range(8):
                p = 128 * qq + 16 * u + lanes
                row = p // H
                col = p % H
                a = (plsc.load_gather(hs_rows, [row, f_half + col]) +
                     plsc.load_gather(ad_rows, [row, col]))
                a = jnp.where(a > 0, a, 0.2 * a)
                plsc.store_scatter(hs_rows, [row, f_half + col], jnp.exp(a))
            return carry

        lax.fori_loop(0, kk * H // 128, exq, 0)

        def group(g, carry):
            kv = g * 16 + lanes
            ws = [plsc.load_gather(hs_rows, [kv, hs]) for hs in hsplats]
            for f in range(f_half):
                fv = zero16 + f
                hv = plsc.load_gather(hs_rows, [kv, fv])
                plsc.store_scatter(hs_rows, [kv, fv], hv * ws[f // fd])
            return carry

        lax.fori_loop(0, kk // 16, group, 0)
        for j in range(kk // KS):
            pltpu.async_copy(hs_rows.at[pl.ds(KS * j, KS)],
                             acc.at[idx_d.at[j]], ssems[b], add=True)

    issue_idx(0, 0)
    issue_idx(1, 1)
    wait_idx(0)
    issue_gath(0)

    def blk(ib, carry):
        for b in range(4):
            i = 4 * ib + b

            @pl.when(i >= 2)
            def _():
                wait_scat((b + 2) % 4)

            @pl.when(i + 2 < n_chunks)
            def _():
                issue_idx((b + 2) % 4, i + 2)

            @pl.when(i + 1 < n_chunks)
            def _():
                wait_idx((b + 1) % 4)
                issue_gath((b + 1) % 4)

            comp(b)
        return carry

    lax.fori_loop(0, n_chunks // 4, blk, 0)
    wait_scat((n_chunks - 2) % 4)
    wait_scat((n_chunks - 1) % 4)

    plsc.subcore_barrier()
    w0 = s * R_OUT
    pltpu.sync_copy(acc.at[pl.ds(w0, R_OUT)],
                    num_o.at[pl.ds(c * ACC_ROWS + w0, R_OUT)])


@functools.lru_cache(maxsize=None)
def _make_sc_edge(f_half, fd, sect=0, kk=K):
    mesh = plsc.VectorSubcoreMesh(core_axis_name="c", subcore_axis_name="s",
                                  num_cores=2, num_subcores=NSUB)
    fw = f_half + H
    buf_set = [
        pltpu.VMEM((kk // KS, KS), jnp.int32),
        pltpu.VMEM((kk // KS, KS), jnp.int32),
        pltpu.VMEM((kk // KS, KS), jnp.int32),
        pltpu.VMEM((kk, H), jnp.float32),
        pltpu.VMEM((kk, fw), jnp.float32),
    ]
    return pl.kernel(
        functools.partial(_sc_edge_body, f_half, fd, sect, kk),
        compiler_params=pltpu.CompilerParams(use_tc_tiling_on_sc=False,
                                             needs_layout_passes=False),
        out_type=[jax.ShapeDtypeStruct((2 * ACC_ROWS, fw), jnp.float32)],
        mesh=mesh,
        scratch_types=(
            [pltpu.VMEM_SHARED((ACC_ROWS, fw), jnp.float32)]
            + buf_set * 4
            + [pltpu.SemaphoreType.DMA] * 12
        ),
    )


def _tc1_body(x_r, w1_r, asm_r, adm_r, h_o, as_o, ad_o, exs_o, numi_o):
    x = x_r[...]
    h = lax.dot_general(x, w1_r[...], (((1,), (1,)), ((), ())),
                        preferred_element_type=jnp.float32)
    a_s = jnp.dot(h, asm_r[...], preferred_element_type=jnp.float32)
    a_d = jnp.dot(h, adm_r[...], preferred_element_type=jnp.float32)
    al = a_s + a_d
    al = jnp.where(al > 0, al, 0.2 * al)
    exs = jnp.exp(al)
    h_o[...] = h
    as_o[...] = a_s
    ad_o[...] = a_d
    exs_o[...] = exs
    numi_o[...] = jnp.concatenate(
        [h[:, F1 * t:F1 * (t + 1)] * exs[:, t:t + 1] for t in range(H)],
        axis=1)


def _tc2_body(num_r, den_r, b1_r, w2_r, asm_r, adm_r,
              h_o, as_o, ad_o, exs_o, numi_o):
    num = num_r[...]
    r = 1.0 / (den_r[...] + 1e-16)
    h1 = jnp.concatenate(
        [num[:, F1 * t:F1 * (t + 1)] * r[:, t:t + 1] for t in range(H)],
        axis=1) + b1_r[...]
    h1 = jnp.where(h1 > 0, h1, jnp.exp(jnp.minimum(h1, 0.0)) - 1.0)
    h = lax.dot_general(h1, w2_r[...], (((1,), (1,)), ((), ())),
                        preferred_element_type=jnp.float32)
    a_s = jnp.dot(h, asm_r[...], preferred_element_type=jnp.float32)
    a_d = jnp.dot(h, adm_r[...], preferred_element_type=jnp.float32)
    al = a_s + a_d
    al = jnp.where(al > 0, al, 0.2 * al)
    exs = jnp.exp(al)
    h_o[...] = h
    as_o[...] = a_s
    ad_o[...] = a_d
    exs_o[...] = exs
    numi_o[...] = jnp.concatenate(
        [h[:, OUT * t:OUT * (t + 1)] * exs[:, t:t + 1] for t in range(H)],
        axis=1)


def _tc3_body(num_r, den_r, b2_r, out_o):
    num = num_r[...]
    r = 0.125 / (den_r[...] + 1e-16)
    acc = num[:, 0:OUT] * r[:, 0:1]
    for t in range(1, H):
        acc = acc + num[:, OUT * t:OUT * (t + 1)] * r[:, t:t + 1]
    out_o[...] = acc + b2_r[...]


_BN = 2000
_GRID = N // _BN


def _row_spec(width):
    return pl.BlockSpec((_BN, width), lambda i: (i, 0))


def _full_spec(shape):
    return pl.BlockSpec(shape, lambda i: tuple(0 for _ in shape))


_tc1 = pl.pallas_call(
    _tc1_body,
    grid=(_GRID,),
    in_specs=[_row_spec(IN), _full_spec((D1, IN)), _full_spec((D1, H)),
              _full_spec((D1, H))],
    out_specs=[_row_spec(D1), _row_spec(H), _row_spec(H), _row_spec(H),
               _row_spec(D1)],
    out_shape=[jax.ShapeDtypeStruct((N, D1), jnp.float32),
               jax.ShapeDtypeStruct((N, H), jnp.float32),
               jax.ShapeDtypeStruct((N, H), jnp.float32),
               jax.ShapeDtypeStruct((N, H), jnp.float32),
               jax.ShapeDtypeStruct((N, D1), jnp.float32)],
)

_tc2 = pl.pallas_call(
    _tc2_body,
    grid=(_GRID,),
    in_specs=[_row_spec(D1), _row_spec(H), _full_spec((1, D1)),
              _full_spec((D2, D1)), _full_spec((D2, H)), _full_spec((D2, H))],
    out_specs=[_row_spec(D2), _row_spec(H), _row_spec(H), _row_spec(H),
               _row_spec(D2)],
    out_shape=[jax.ShapeDtypeStruct((N, D2), jnp.float32),
               jax.ShapeDtypeStruct((N, H), jnp.float32),
               jax.ShapeDtypeStruct((N, H), jnp.float32),
               jax.ShapeDtypeStruct((N, H), jnp.float32),
               jax.ShapeDtypeStruct((N, D2), jnp.float32)],
)

_tc3 = pl.pallas_call(
    _tc3_body,
    grid=(_GRID,),
    in_specs=[_row_spec(D2), _row_spec(H), _full_spec((1, OUT))],
    out_specs=_row_spec(OUT),
    out_shape=jax.ShapeDtypeStruct((N, OUT), jnp.float32),
)


def _expand_att(a):
    hh, ff = a.shape
    eye = jnp.eye(hh, dtype=jnp.float32)
    return (a[:, :, None] * eye[:, None, :]).reshape(hh * ff, hh)


def kernel(x, edge, W1, a_src1, a_dst1, b1, W2, a_src2, a_dst2, b2):
    As1 = _expand_att(a_src1)
    Ad1 = _expand_att(a_dst1)
    As2 = _expand_att(a_src2)
    Ad2 = _expand_att(a_dst2)
    src = jnp.concatenate(
        [edge[0].astype(jnp.int32),
         jnp.zeros((EP - E,), jnp.int32)]).reshape(EP // KS, KS)
    dst = jnp.concatenate(
        [edge[1].astype(jnp.int32),
         jnp.full((EP - E,), N, jnp.int32)]).reshape(EP // KS, KS)

    h1, as1, ad1, exs1, numi1 = _tc1(x, W1, As1, Ad1)
    fh1 = D1 // 2
    ht1 = jnp.concatenate([
        jnp.concatenate([h1[:, :fh1], as1], axis=1),
        jnp.concatenate([h1[:, fh1:], as1], axis=1)], axis=0)
    ad1p = jnp.concatenate([ad1, jnp.zeros((1, H), jnp.float32)], axis=0)
    pad1 = jnp.zeros((ACC_ROWS - N, fh1 + H), jnp.float32)
    numi1f = jnp.concatenate([
        jnp.concatenate([numi1[:, :fh1], exs1], axis=1), pad1,
        jnp.concatenate([numi1[:, fh1:], exs1], axis=1), pad1], axis=0)
    out1 = _make_sc_edge(fh1, F1, 0, 512)(src, dst, ht1, ad1p, numi1f)
    num1 = out1[0] if isinstance(out1, (tuple, list)) else out1
    num1c = jnp.concatenate(
        [num1[:N, :fh1], num1[ACC_ROWS:ACC_ROWS + N, :fh1]], axis=1)
    den1 = num1[:N, fh1:]

    h2, as2, ad2, exs2, numi2 = _tc2(num1c, den1, b1.reshape(1, D1),
                                     W2, As2, Ad2)
    ad2p = jnp.concatenate([ad2, jnp.zeros((1, H), jnp.float32)], axis=0)
    fq = D2 // 4
    padq = jnp.zeros((ACC_ROWS - N, fq + H), jnp.float32)
    quarters = []
    den2 = None
    for p in range(2):
        ht = jnp.concatenate([
            jnp.concatenate([h2[:, 2 * p * fq:(2 * p + 1) * fq], as2],
                            axis=1),
            jnp.concatenate([h2[:, (2 * p + 1) * fq:(2 * p + 2) * fq], as2],
                            axis=1)], axis=0)
        numib = jnp.concatenate([
            jnp.concatenate([numi2[:, 2 * p * fq:(2 * p + 1) * fq], exs2],
                            axis=1), padq,
            jnp.concatenate([numi2[:, (2 * p + 1) * fq:(2 * p + 2) * fq],
                             exs2], axis=1), padq], axis=0)
        outq = _make_sc_edge(fq, OUT, p, KS)(src, dst, ht, ad2p, numib)
        numq = outq[0] if isinstance(outq, (tuple, list)) else outq
        quarters += [numq[:N, :fq], numq[ACC_ROWS:ACC_ROWS + N, :fq]]
        if p == 0:
            den2 = numq[:N, fq:]
    num2c = jnp.concatenate(quarters, axis=1)

    return _tc3(num2c, den2, b2.reshape(1, OUT))
